# baseline (speedup 1.0000x reference)
import functools
import jax
import jax.numpy as jnp
from jax.experimental import pallas as pl
from jax.experimental.pallas import tpu as pltpu

_NUM_EXPERTS = 8
_BLOCK_ROWS = 4096


def _router_block(hs_ref, wt_ref, bias_ref, scores_ref, idx_ref):
    x = hs_ref[...]
    wt = wt_ref[...]
    logits = jax.lax.dot_general(
        x, wt, (((1,), (0,)), ((), ())),
        preferred_element_type=jnp.float32,
    )
    logits = logits + bias_ref[...]
    r, e = logits.shape

    lane = jax.lax.broadcasted_iota(jnp.int32, (r, e), 1)
    bits = jax.lax.bitcast_convert_type(logits, jnp.int32)
    key = jax.lax.bitcast_convert_type((bits & -8) | (7 - lane), jnp.float32)

    m1 = jnp.max(key, axis=1, keepdims=True)
    is1 = key == m1
    m2 = jnp.max(jnp.where(is1, -jnp.inf, key), axis=1, keepdims=True)
    is2 = key == m2

    m1b = jax.lax.bitcast_convert_type(m1, jnp.int32)
    m2b = jax.lax.bitcast_convert_type(m2, jnp.int32)
    v1 = jax.lax.bitcast_convert_type(m1b & -8, jnp.float32)
    v2 = jax.lax.bitcast_convert_type(m2b & -8, jnp.float32)

    z = jnp.exp(v2 - v1)
    s1 = 1.0 / (1.0 + z)
    s2 = z * s1

    scores_ref[...] = jnp.where(is1, s1, jnp.where(is2, s2, 0.0))
    idx_ref[...] = jnp.concatenate(
        [7 - (m1b & 7), 7 - (m2b & 7)], axis=1)


@jax.jit
def kernel(hidden_states, weight, bias):
    hidden = weight.shape[1]
    hs = hidden_states.reshape(-1, hidden)
    n = hs.shape[0]
    e = weight.shape[0]
    grid = (n // _BLOCK_ROWS,)

    scores, indices = pl.pallas_call(
        _router_block,
        grid=grid,
        in_specs=[
            pl.BlockSpec((_BLOCK_ROWS, hidden), lambda i: (i, 0)),
            pl.BlockSpec((hidden, e), lambda i: (0, 0)),
            pl.BlockSpec((1, e), lambda i: (0, 0)),
        ],
        out_specs=[
            pl.BlockSpec((_BLOCK_ROWS, e), lambda i: (i, 0)),
            pl.BlockSpec((_BLOCK_ROWS, 2), lambda i: (i, 0)),
        ],
        out_shape=[
            jax.ShapeDtypeStruct((n, e), jnp.float32),
            jax.ShapeDtypeStruct((n, 2), jnp.int32),
        ],
        compiler_params=pltpu.CompilerParams(
            dimension_semantics=("arbitrary",),
        ),
    )(hs, weight.T, bias.reshape(1, e))
    return scores, indices
